# baseline (speedup 1.0000x reference)
import functools
import math

import jax
import jax.numpy as jnp
from jax import lax
from jax.experimental import pallas as pl
from jax.experimental.pallas import tpu as pltpu
from jax.experimental.pallas import tpu_sc as plsc

C = 192
D3, H3, W3 = 8, 48, 48
N = D3 * H3 * W3
QK = 192
MLP = 384
HEADS = 6
HD = QK // HEADS
NUM_TOKENS = 8
GS = 128
NG = N // GS
CHUNK = N // NUM_TOKENS + 1
PAD_START = N - NUM_TOKENS

SLAB = H3 * W3
APR = 128
PSLAB = SLAB + 2 * APR
QR = 288

NW = 32
ROWS_W = N // NW
SC_CH = 96
SC_NCH = ROWS_W // SC_CH


def _k1_body(x_ref, g_ref, b_ref, wkg_ref, wvg_ref,
             xn_ref, means_ref, kg_ref, vg_ref, acc_ref):
    b = pl.program_id(0)
    xb = x_ref[...]
    mu = jnp.mean(xb, axis=0, keepdims=True)
    var = jnp.mean((xb - mu) ** 2, axis=0, keepdims=True)
    xnb = (xb - mu) * lax.rsqrt(var + 1e-5) * g_ref[...] + b_ref[...]
    xnT = xnb.T
    xn_ref[...] = xnT

    @pl.when(b == 0)
    def _():
        acc_ref[...] = jnp.zeros_like(acc_ref)

    n = b * GS + lax.broadcasted_iota(jnp.int32, (1, GS), 1)
    ci = lax.broadcasted_iota(jnp.int32, (NUM_TOKENS, GS), 0)
    cn = n // CHUNK
    oh = (ci == cn).astype(jnp.float32)
    oh = oh + ((ci == NUM_TOKENS - 1) & (n >= PAD_START)).astype(jnp.float32)
    acc_ref[...] += jnp.dot(oh, xnT, preferred_element_type=jnp.float32,
                            precision=lax.Precision.HIGHEST)

    @pl.when(b == NG - 1)
    def _():
        means = acc_ref[...] / float(CHUNK)
        means_ref[...] = means
        kg_ref[...] = jnp.dot(means, wkg_ref[...],
                              preferred_element_type=jnp.float32)
        vg_ref[...] = jnp.dot(means, wvg_ref[...],
                              preferred_element_type=jnp.float32)


def _run_k1(x2d, norm_g, norm_b, wk_g, wv_g):
    return pl.pallas_call(
        _k1_body,
        grid=(NG,),
        in_specs=[
            pl.BlockSpec((C, GS), lambda b: (0, b)),
            pl.BlockSpec((C, 1), lambda b: (0, 0)),
            pl.BlockSpec((C, 1), lambda b: (0, 0)),
            pl.BlockSpec((C, QK), lambda b: (0, 0)),
            pl.BlockSpec((C, C), lambda b: (0, 0)),
        ],
        out_specs=[
            pl.BlockSpec((GS, C), lambda b: (b, 0)),
            pl.BlockSpec((NUM_TOKENS, C), lambda b: (0, 0)),
            pl.BlockSpec((NUM_TOKENS, QK), lambda b: (0, 0)),
            pl.BlockSpec((NUM_TOKENS, C), lambda b: (0, 0)),
        ],
        out_shape=[
            jax.ShapeDtypeStruct((N, C), jnp.float32),
            jax.ShapeDtypeStruct((NUM_TOKENS, C), jnp.float32),
            jax.ShapeDtypeStruct((NUM_TOKENS, QK), jnp.float32),
            jax.ShapeDtypeStruct((NUM_TOKENS, C), jnp.float32),
        ],
        scratch_shapes=[pltpu.VMEM((NUM_TOKENS, C), jnp.float32)],
    )(x2d, norm_g.reshape(C, 1), norm_b.reshape(C, 1), wk_g, wv_g)


def _k2_body(x_ref, g_ref, b_ref, means_ref, out_ref, totals, base, run):
    p = pl.program_id(0)
    b = pl.program_id(1)

    @pl.when((p == 0) & (b == 0))
    def _():
        for c in range(NUM_TOKENS):
            totals[c] = 0.0

    xb = x_ref[...]
    mu = jnp.mean(xb, axis=0, keepdims=True)
    var = jnp.mean((xb - mu) ** 2, axis=0, keepdims=True)
    xnb = (xb - mu) * lax.rsqrt(var + 1e-5) * g_ref[...] + b_ref[...]
    mn = means_ref[...]
    nrm = jnp.sqrt(jnp.sum(mn * mn, axis=1, keepdims=True))
    mnn = mn / jnp.maximum(nrm, 1e-12)
    tnrm = jnp.sqrt(jnp.sum(xnb * xnb, axis=0, keepdims=True))
    xnn = xnb / jnp.maximum(tnrm, 1e-12)
    S = jnp.dot(mnn.astype(jnp.bfloat16), xnn.astype(jnp.bfloat16),
                preferred_element_type=jnp.float32)
    m = jnp.max(S, axis=0, keepdims=True)
    ci = lax.broadcasted_iota(jnp.int32, (NUM_TOKENS, GS), 0)
    cl = jnp.min(jnp.where(S >= m, ci, NUM_TOKENS), axis=0, keepdims=True)

    @pl.when(p == 0)
    def _():
        for c in range(NUM_TOKENS):
            totals[c] += jnp.sum((cl == c).astype(jnp.float32))

    @pl.when((p == 1) & (b == 0))
    def _():
        base[0] = 0.0
        for c in range(1, NUM_TOKENS):
            base[c] = base[c - 1] + totals[c - 1]
        for c in range(NUM_TOKENS):
            run[c] = 0.0

    @pl.when(p == 1)
    def _():
        tri = (lax.broadcasted_iota(jnp.int32, (GS, GS), 0)
               < lax.broadcasted_iota(jnp.int32, (GS, GS), 1)).astype(jnp.float32)
        pos = jnp.zeros((1, GS), jnp.float32)
        for c in range(NUM_TOKENS):
            eq = (cl == c).astype(jnp.float32)
            pre = jnp.dot(eq, tri, preferred_element_type=jnp.float32)
            pos = pos + eq * (base[c] + run[c] + pre)
            run[c] += jnp.sum(eq)
        out_ref[...] = pos.astype(jnp.int32).reshape(1, 1, GS)


def _run_k2(x2d, norm_g, norm_b, means):
    return pl.pallas_call(
        _k2_body,
        grid=(2, NG),
        in_specs=[
            pl.BlockSpec((C, GS), lambda p, b: (0, b)),
            pl.BlockSpec((C, 1), lambda p, b: (0, 0)),
            pl.BlockSpec((C, 1), lambda p, b: (0, 0)),
            pl.BlockSpec((NUM_TOKENS, C), lambda p, b: (0, 0)),
        ],
        out_specs=pl.BlockSpec((1, 1, GS), lambda p, b: (b * p, 0, 0)),
        out_shape=jax.ShapeDtypeStruct((NG, 1, GS), jnp.int32),
        scratch_shapes=[
            pltpu.SMEM((NUM_TOKENS,), jnp.float32),
            pltpu.SMEM((NUM_TOKENS,), jnp.float32),
            pltpu.SMEM((NUM_TOKENS,), jnp.float32),
        ],
    )(x2d, norm_g.reshape(C, 1), norm_b.reshape(C, 1), means)


def _sc_scatter_body(xn_hbm, iperm_hbm, out_hbm, idx_v, rows_v, sem):
    cc = lax.axis_index("c")
    ss = lax.axis_index("s")
    wid = ss * 2 + cc
    base = wid * ROWS_W
    pltpu.sync_copy(iperm_hbm.at[wid], idx_v)
    for j in range(SC_NCH):
        pltpu.sync_copy(xn_hbm.at[pl.ds(base + j * SC_CH, SC_CH)], rows_v)
        pltpu.async_copy(rows_v, out_hbm.at[idx_v.at[j]], sem).wait()


def _run_sc_scatter(xn, iperm3):
    mesh = plsc.VectorSubcoreMesh(core_axis_name="c", subcore_axis_name="s")
    fn = functools.partial(
        pl.kernel,
        mesh=mesh,
        out_type=jax.ShapeDtypeStruct((N, C), jnp.float32),
        scratch_types=[
            pltpu.VMEM((SC_NCH, SC_CH), jnp.int32),
            pltpu.VMEM((SC_CH, C), jnp.float32),
            pltpu.SemaphoreType.DMA,
        ],
        compiler_params=pltpu.CompilerParams(use_tc_tiling_on_sc=False),
    )(_sc_scatter_body)
    return fn(xn, iperm3)


def _sc_gather_body(ys_hbm, iperm_hbm, out_hbm, idx_v, rows_v, sem):
    cc = lax.axis_index("c")
    ss = lax.axis_index("s")
    wid = ss * 2 + cc
    base = wid * ROWS_W
    pltpu.sync_copy(iperm_hbm.at[wid], idx_v)
    for j in range(SC_NCH):
        pltpu.async_copy(ys_hbm.at[idx_v.at[j]], rows_v, sem).wait()
        pltpu.sync_copy(rows_v, out_hbm.at[pl.ds(base + j * SC_CH, SC_CH)])


def _run_sc_gather(ys, iperm3):
    mesh = plsc.VectorSubcoreMesh(core_axis_name="c", subcore_axis_name="s")
    fn = functools.partial(
        pl.kernel,
        mesh=mesh,
        out_type=jax.ShapeDtypeStruct((N, C), jnp.float32),
        scratch_types=[
            pltpu.VMEM((SC_NCH, SC_CH), jnp.int32),
            pltpu.VMEM((SC_CH, C), jnp.float32),
            pltpu.SemaphoreType.DMA,
        ],
        compiler_params=pltpu.CompilerParams(use_tc_tiling_on_sc=False),
    )(_sc_gather_body)
    return fn(ys, iperm3)


def _k4_body(xs_ref, wq_ref, wk_ref, wv_ref, q_ref, k_ref, v_ref):
    xb = xs_ref[...]
    q_ref[...] = jnp.dot(xb, wq_ref[...], preferred_element_type=jnp.float32)
    k_ref[...] = jnp.dot(xb, wk_ref[...], preferred_element_type=jnp.float32)
    v_ref[...] = jnp.dot(xb, wv_ref[...], preferred_element_type=jnp.float32)


def _run_k4(xs, wq, wk, wv):
    return pl.pallas_call(
        _k4_body,
        grid=(NG,),
        in_specs=[
            pl.BlockSpec((GS, C), lambda b: (b, 0)),
            pl.BlockSpec((C, QK), lambda b: (0, 0)),
            pl.BlockSpec((C, QK), lambda b: (0, 0)),
            pl.BlockSpec((C, C), lambda b: (0, 0)),
        ],
        out_specs=[
            pl.BlockSpec((GS, QK), lambda b: (b, 0)),
            pl.BlockSpec((GS, QK), lambda b: (b, 0)),
            pl.BlockSpec((GS, C), lambda b: (b, 0)),
        ],
        out_shape=[
            jax.ShapeDtypeStruct((N, QK), jnp.float32),
            jax.ShapeDtypeStruct((N, QK), jnp.float32),
            jax.ShapeDtypeStruct((N, C), jnp.float32),
        ],
    )(xs, wq, wk, wv)


def _k5_body(q_ref, ka_ref, kb_ref, va_ref, vb_ref, kg_ref, vg_ref,
             wp_ref, cw_ref, y_ref, k2s, v2s):
    b = pl.program_id(0)
    last = b == NG - 1

    @pl.when(last)
    def _():
        flip = ((lax.broadcasted_iota(jnp.int32, (GS, GS), 0)
                 + lax.broadcasted_iota(jnp.int32, (GS, GS), 1))
                == GS - 1).astype(jnp.float32)
        k2s[...] = jnp.dot(flip, ka_ref[...], preferred_element_type=jnp.float32)
        v2s[...] = jnp.dot(flip, va_ref[...], preferred_element_type=jnp.float32)

    @pl.when(jnp.logical_not(last))
    def _():
        k2s[...] = kb_ref[...]
        v2s[...] = vb_ref[...]

    q = q_ref[...]
    ka = ka_ref[...]
    va = va_ref[...]
    kb = k2s[...]
    vb = v2s[...]
    kg = kg_ref[...]
    vg = vg_ref[...]
    scale = 1.0 / math.sqrt(HD)

    outs = []
    for h in range(HEADS):
        sl = slice(h * HD, (h + 1) * HD)
        qh = q[:, sl]
        s1a = jnp.dot(qh, ka[:, sl].T, preferred_element_type=jnp.float32) * scale
        s1b = jnp.dot(qh, kb[:, sl].T, preferred_element_type=jnp.float32) * scale
        m = jnp.maximum(jnp.max(s1a, axis=1, keepdims=True),
                        jnp.max(s1b, axis=1, keepdims=True))
        ea = jnp.exp(s1a - m)
        eb = jnp.exp(s1b - m)
        den = jnp.sum(ea, axis=1, keepdims=True) + jnp.sum(eb, axis=1, keepdims=True)
        o1 = (jnp.dot(ea, va[:, sl], preferred_element_type=jnp.float32)
              + jnp.dot(eb, vb[:, sl], preferred_element_type=jnp.float32)) / den
        s2 = jnp.dot(qh, kg[:, sl].T, preferred_element_type=jnp.float32) * scale
        m2 = jnp.max(s2, axis=1, keepdims=True)
        e2 = jnp.exp(s2 - m2)
        o2 = jnp.dot(e2, vg[:, sl], preferred_element_type=jnp.float32) \
            / jnp.sum(e2, axis=1, keepdims=True)
        outs.append(o1 + o2)
    out = jnp.concatenate(outs, axis=1)
    y = jnp.dot(out, wp_ref[...], preferred_element_type=jnp.float32)
    y = lax.dot_general(y, cw_ref[...], (((1,), (1,)), ((), ())),
                        preferred_element_type=jnp.float32)
    y_ref[...] = y


def _run_k5(q, k, v, kg, vg, wproj, cw):
    nxt = lambda b: (jnp.minimum(b + 1, NG - 1), 0)
    return pl.pallas_call(
        _k5_body,
        grid=(NG,),
        in_specs=[
            pl.BlockSpec((GS, QK), lambda b: (b, 0)),
            pl.BlockSpec((GS, QK), lambda b: (b, 0)),
            pl.BlockSpec((GS, QK), nxt),
            pl.BlockSpec((GS, C), lambda b: (b, 0)),
            pl.BlockSpec((GS, C), nxt),
            pl.BlockSpec((NUM_TOKENS, QK), lambda b: (0, 0)),
            pl.BlockSpec((NUM_TOKENS, C), lambda b: (0, 0)),
            pl.BlockSpec((C, C), lambda b: (0, 0)),
            pl.BlockSpec((C, C), lambda b: (0, 0)),
        ],
        out_specs=pl.BlockSpec((GS, C), lambda b: (b, 0)),
        out_shape=jax.ShapeDtypeStruct((N, C), jnp.float32),
        scratch_shapes=[
            pltpu.VMEM((GS, QK), jnp.float32),
            pltpu.VMEM((GS, C), jnp.float32),
        ],
    )(q, k, k, v, v, kg, vg, wproj, cw)


def _k7_body(x_ref, y_ref, g_ref, b_ref, w1_ref, b1_ref, x2_ref, a_ref):
    xt = x_ref[...].T
    x2 = xt + y_ref[...]
    x2_ref[...] = x2
    mu = jnp.mean(x2, axis=1, keepdims=True)
    var = jnp.mean((x2 - mu) ** 2, axis=1, keepdims=True)
    zn = (x2 - mu) * lax.rsqrt(var + 1e-5) * g_ref[...] + b_ref[...]
    h1 = jnp.dot(zn, w1_ref[...], preferred_element_type=jnp.float32) + b1_ref[...]
    a_ref[...] = 0.5 * h1 * (1.0 + lax.erf(h1 * (1.0 / math.sqrt(2.0))))


def _run_k7(x2d, y_orig, mlp_ng, mlp_nb, fc1_w, fc1_b):
    return pl.pallas_call(
        _k7_body,
        grid=(NG,),
        in_specs=[
            pl.BlockSpec((C, GS), lambda b: (0, b)),
            pl.BlockSpec((GS, C), lambda b: (b, 0)),
            pl.BlockSpec((1, C), lambda b: (0, 0)),
            pl.BlockSpec((1, C), lambda b: (0, 0)),
            pl.BlockSpec((C, MLP), lambda b: (0, 0)),
            pl.BlockSpec((1, MLP), lambda b: (0, 0)),
        ],
        out_specs=[
            pl.BlockSpec((GS, C), lambda b: (b, 0)),
            pl.BlockSpec((GS, MLP),
                         lambda b: ((b // 18) * 20 + 1 + (b % 18), 0)),
        ],
        out_shape=[
            jax.ShapeDtypeStruct((N, C), jnp.float32),
            jax.ShapeDtypeStruct((D3 * PSLAB, MLP), jnp.float32),
        ],
    )(x2d, y_orig, mlp_ng.reshape(1, C), mlp_nb.reshape(1, C),
      fc1_w, fc1_b.reshape(1, MLP))


def _k8_body(s0, s1, s2, s3, s4, x2_ref, wdw_ref, bdw_ref, w2_ref, b2_ref,
             out_ref):
    d = pl.program_id(0)
    slabs = [s0, s1, s2, s3, s4]
    okf = []
    for dz in range(-2, 3):
        ok_d = (d + dz >= 0) & (d + dz <= D3 - 1)
        okf.append(jnp.where(ok_d, 1.0, 0.0))
    for c0 in range(0, SLAB, QR):
        rr = lax.broadcasted_iota(jnp.int32, (QR, 1), 0) + c0
        hh = rr // W3
        ww = rr % W3
        acc = jnp.zeros((QR, MLP), jnp.float32)
        for iz, dz in enumerate(range(-2, 3)):
            slab = slabs[iz]
            for dh in range(-2, 3):
                for dw_ in range(-2, 3):
                    tap = iz * 25 + (dh + 2) * 5 + (dw_ + 2)
                    off = dh * W3 + dw_
                    mk = ((hh + dh >= 0) & (hh + dh < H3)
                          & (ww + dw_ >= 0) & (ww + dw_ < W3))
                    wrow = wdw_ref[tap:tap + 1, :] * okf[iz]
                    src = slab[APR + c0 + off:APR + c0 + off + QR, :]
                    acc = acc + jnp.where(mk, src * wrow, 0.0)
        zc = acc + bdw_ref[...]
        zc = 0.5 * zc * (1.0 + lax.erf(zc * (1.0 / math.sqrt(2.0))))
        s = s2[APR + c0:APR + c0 + QR, :] + zc
        o = jnp.dot(s, w2_ref[...], preferred_element_type=jnp.float32) \
            + b2_ref[...] + x2_ref[c0:c0 + QR, :]
        out_ref[:, c0:c0 + QR] = o.T


def _run_k8(a, x2, dwp, dw_b, fc2_w, fc2_b):
    def slab_spec(dz):
        return pl.BlockSpec(
            (PSLAB, MLP),
            lambda d, dz=dz: (jnp.clip(d + dz, 0, D3 - 1), 0))
    return pl.pallas_call(
        _k8_body,
        grid=(D3,),
        in_specs=[
            slab_spec(-2), slab_spec(-1), slab_spec(0), slab_spec(1), slab_spec(2),
            pl.BlockSpec((SLAB, C), lambda d: (d, 0)),
            pl.BlockSpec((128, MLP), lambda d: (0, 0)),
            pl.BlockSpec((1, MLP), lambda d: (0, 0)),
            pl.BlockSpec((MLP, C), lambda d: (0, 0)),
            pl.BlockSpec((1, C), lambda d: (0, 0)),
        ],
        out_specs=pl.BlockSpec((C, SLAB), lambda d: (0, d)),
        out_shape=jax.ShapeDtypeStruct((C, N), jnp.float32),
    )(a, a, a, a, a, x2, dwp, dw_b.reshape(1, MLP), fc2_w, fc2_b.reshape(1, C))


def kernel(x, norm_g, norm_b, wq, wk, wv, wproj, wk_g, wv_g, conv1x1_w,
           mlp_ng, mlp_nb, fc1_w, fc1_b, dw_w, dw_b, fc2_w, fc2_b):
    x2d = x.reshape(C, N)

    xn, means, kg, vg = _run_k1(x2d, norm_g, norm_b, wk_g, wv_g)
    iperm = _run_k2(x2d, norm_g, norm_b, means)
    iperm3 = iperm.reshape(NW, SC_NCH, SC_CH)

    xs = _run_sc_scatter(xn, iperm3)
    q, k, v = _run_k4(xs, wq, wk, wv)
    ys = _run_k5(q, k, v, kg, vg, wproj, conv1x1_w)
    y_orig = _run_sc_gather(ys, iperm3)

    x2, a = _run_k7(x2d, y_orig, mlp_ng, mlp_nb, fc1_w, fc1_b)

    dwp = jnp.concatenate(
        [dw_w.reshape(MLP, 125).T, jnp.zeros((3, MLP), jnp.float32)], axis=0)
    out = _run_k8(a, x2, dwp, dw_b, fc2_w, fc2_b)
    return out.reshape(1, C, D3, H3, W3)
